# baseline (speedup 1.0000x reference)
import functools

import jax
import jax.numpy as jnp
from jax import lax
from jax.experimental import pallas as pl
from jax.experimental.pallas import tpu as pltpu
from jax.experimental.pallas import tpu_sc as plsc

N = 10000
D = 128
E = 320000
NC, NS, L = 2, 16, 16
NW = NC * NS
DW = 144
NP = 10016
CB = 96
NIT0 = 34
NIT1 = 19
NCHMAX = 4 * NIT0
RPT = NP // NS
BN = 2000


def _dense_body(x_ref, wsw_ref, wsb_ref, wnw_ref, wnb_ref, atw_ref, atb_ref,
                xs_ref, xne_ref, ab_ref):
    xb = x_ref[...]
    xs_ref[...] = jnp.dot(xb, wsw_ref[...],
                          preferred_element_type=jnp.float32) + wsb_ref[...]
    xn = jnp.dot(xb, wnw_ref[...],
                 preferred_element_type=jnp.float32) + wnb_ref[...]
    atw = atw_ref[...]
    a = jnp.dot(xb, atw[:D, :], preferred_element_type=jnp.float32)
    b = jnp.dot(xn, atw[D:, :], preferred_element_type=jnp.float32) \
        + atb_ref[...]
    m = x_ref.shape[0]
    xne_ref[...] = jnp.concatenate(
        [xn, jnp.ones((m, 1), jnp.float32), b,
         jnp.zeros((m, DW - D - 2), jnp.float32)], axis=-1)
    ab_ref[...] = jnp.concatenate([a, b], axis=-1)


def _dense(x, ws_w, ws_b, wn_w, wn_b, at_w, at_b):
    grid = (N // BN,)
    return pl.pallas_call(
        _dense_body,
        grid=grid,
        in_specs=[
            pl.BlockSpec((BN, D), lambda i: (i, 0)),
            pl.BlockSpec((D, D), lambda i: (0, 0)),
            pl.BlockSpec((D,), lambda i: (0,)),
            pl.BlockSpec((D, D), lambda i: (0, 0)),
            pl.BlockSpec((D,), lambda i: (0,)),
            pl.BlockSpec((2 * D, 1), lambda i: (0, 0)),
            pl.BlockSpec((1,), lambda i: (0,)),
        ],
        out_specs=[
            pl.BlockSpec((BN, D), lambda i: (i, 0)),
            pl.BlockSpec((BN, DW), lambda i: (i, 0)),
            pl.BlockSpec((BN, 2), lambda i: (i, 0)),
        ],
        out_shape=[
            jax.ShapeDtypeStruct((N, D), jnp.float32),
            jax.ShapeDtypeStruct((N, DW), jnp.float32),
            jax.ShapeDtypeStruct((N, 2), jnp.float32),
        ],
    )(x, ws_w, ws_b, wn_w, wn_b, at_w, at_b)


def _sc_body(xne_hbm, row_hbm, col_hbm, a_hbm, z_hbm, out_hbm,
             rowc0, colc0, rowc1, colc1, rowc2, colc2, rowc3, colc3,
             av, rv0, rv1, aggsh,
             sem_i0, sem_i1, sem_i2, sem_i3, sem_g0, sem_g1,
             sem_s0, sem_s1):
    cid = lax.axis_index("c")
    sid = lax.axis_index("s")
    wid = cid * NS + sid
    rowc = [rowc0, rowc1, rowc2, rowc3]
    colc = [colc0, colc1, colc2, colc3]
    rv = [rv0, rv1]
    sem_i = [sem_i0, sem_i1, sem_i2, sem_i3]
    sem_g = [sem_g0, sem_g1]
    sem_s = [sem_s0, sem_s1]
    rhb = row_hbm.at[wid]
    chb = col_hbm.at[wid]
    nit = jnp.where(cid == 0, NIT0, NIT1)

    pltpu.sync_copy(a_hbm, av)
    pltpu.sync_copy(z_hbm.at[pl.ds(sid * RPT, RPT)],
                    aggsh.at[pl.ds(sid * RPT, RPT)])
    plsc.subcore_barrier()

    pltpu.sync_copy(rhb.at[0], rowc[0])
    pltpu.sync_copy(chb.at[0], colc[0])
    pltpu.async_copy(rhb.at[1], rowc[1], sem_i[1])
    pltpu.async_copy(chb.at[1], colc[1], sem_i[1])
    pltpu.async_copy(xne_hbm.at[rowc[0]], rv[0], sem_g[0])

    def compute(q, p):
        def group(g, cy):
            c16 = colc[q][pl.ds(g * L, L)]
            jrow = g * L + lax.iota(jnp.int32, L)
            jcol = jnp.full((L,), D + 1, jnp.int32)
            b16 = plsc.load_gather(rv[p], [jrow, jcol])
            z = plsc.load_gather(av, [c16]) + b16
            s = 1.0 / (1.0 + jnp.exp(-z))
            for jl in range(L):
                j = g * L + jl
                sj = s[jl]
                for u in range(DW // L):
                    rv[p][j, pl.ds(u * L, L)] = rv[p][j, pl.ds(u * L, L)] * sj
            return cy
        lax.fori_loop(0, CB // L, group, 0)

    def iter_body(m, cy):
        for r in range(4):
            c = 4 * m + r
            q, p = r, r % 2
            qn1, qn2, qm1 = (r + 1) % 4, (r + 2) % 4, (r + 3) % 4

            def fire_idx(c=c, qn2=qn2):
                pltpu.async_copy(rhb.at[c + 2], rowc[qn2], sem_i[qn2])
                pltpu.async_copy(chb.at[c + 2], colc[qn2], sem_i[qn2])

            if r >= 2:
                pl.when(m < nit - 1)(fire_idx)
            else:
                fire_idx()

            pltpu.make_async_copy(xne_hbm.at[rowc[q]], rv[p], sem_g[p]).wait()
            compute(q, p)

            def wait_sc(p=p, qm1=qm1):
                pltpu.make_async_copy(rv[1 - p], aggsh.at[colc[qm1]],
                                      sem_s[1 - p]).wait()

            if r == 0:
                pl.when(m > 0)(wait_sc)
            else:
                wait_sc()

            def fire_g(c=c, qn1=qn1, p=p):
                pltpu.make_async_copy(rhb.at[c + 1], rowc[qn1],
                                      sem_i[qn1]).wait()
                pltpu.make_async_copy(chb.at[c + 1], colc[qn1],
                                      sem_i[qn1]).wait()
                pltpu.async_copy(xne_hbm.at[rowc[qn1]], rv[1 - p],
                                 sem_g[1 - p])

            if r == 3:
                pl.when(m < nit - 1)(fire_g)
            else:
                fire_g()

            pltpu.async_copy(rv[p], aggsh.at[colc[q]], sem_s[p], add=True)
        return cy

    lax.fori_loop(0, nit, iter_body, 0)
    pltpu.make_async_copy(rv[1], aggsh.at[colc[3]], sem_s[1]).wait()
    plsc.subcore_barrier()
    pltpu.sync_copy(aggsh.at[pl.ds(sid * RPT, RPT)],
                    out_hbm.at[cid].at[pl.ds(sid * RPT, RPT)])


_sc_agg = functools.partial(
    pl.kernel,
    _sc_body,
    out_type=jax.ShapeDtypeStruct((NC, NP, DW), jnp.float32),
    mesh=plsc.VectorSubcoreMesh(core_axis_name="c", subcore_axis_name="s"),
    scratch_types=[
        pltpu.VMEM((CB,), jnp.int32),
        pltpu.VMEM((CB,), jnp.int32),
        pltpu.VMEM((CB,), jnp.int32),
        pltpu.VMEM((CB,), jnp.int32),
        pltpu.VMEM((CB,), jnp.int32),
        pltpu.VMEM((CB,), jnp.int32),
        pltpu.VMEM((CB,), jnp.int32),
        pltpu.VMEM((CB,), jnp.int32),
        pltpu.VMEM((NP,), jnp.float32),
        pltpu.VMEM((CB, DW), jnp.float32),
        pltpu.VMEM((CB, DW), jnp.float32),
        pltpu.VMEM_SHARED((NP, DW), jnp.float32),
        pltpu.SemaphoreType.DMA,
        pltpu.SemaphoreType.DMA,
        pltpu.SemaphoreType.DMA,
        pltpu.SemaphoreType.DMA,
        pltpu.SemaphoreType.DMA,
        pltpu.SemaphoreType.DMA,
        pltpu.SemaphoreType.DMA,
        pltpu.SemaphoreType.DMA,
    ],
    compiler_params=pltpu.CompilerParams(needs_layout_passes=False,
                                         use_tc_tiling_on_sc=False),
)()


def _epi_body(x_ref, xs_ref, p0_ref, p1_ref, g_ref, bb_ref, o_ref):
    agg = p0_ref[:, :D] + p1_ref[:, :D]
    asum = p0_ref[:, D:D + 1] + p1_ref[:, D:D + 1]
    u = xs_ref[...] + agg / (asum + 1e-6)
    h = 0.5 * u * (1.0 + lax.erf(u * (1.0 / jnp.sqrt(2.0)))) + x_ref[...]
    m = jnp.mean(h, axis=-1, keepdims=True)
    v = jnp.mean((h - m) ** 2, axis=-1, keepdims=True)
    o_ref[...] = (h - m) / jnp.sqrt(v + 1e-5) * g_ref[...] + bb_ref[...]


def _epilogue(x, xs, p0, p1, ln_g, ln_b):
    grid = (N // BN,)
    return pl.pallas_call(
        _epi_body,
        grid=grid,
        in_specs=[
            pl.BlockSpec((BN, D), lambda i: (i, 0)),
            pl.BlockSpec((BN, D), lambda i: (i, 0)),
            pl.BlockSpec((BN, DW), lambda i: (i, 0)),
            pl.BlockSpec((BN, DW), lambda i: (i, 0)),
            pl.BlockSpec((D,), lambda i: (0,)),
            pl.BlockSpec((D,), lambda i: (0,)),
        ],
        out_specs=pl.BlockSpec((BN, D), lambda i: (i, 0)),
        out_shape=jax.ShapeDtypeStruct((N, D), jnp.float32),
    )(x, xs, p0, p1, ln_g, ln_b)


def kernel(x, edge_index, ws_w0, ws_b0, wn_w0, wn_b0, at_w0, at_b0, ln_g0,
           ln_b0, ws_w1, ws_b1, wn_w1, wn_b1, at_w1, at_b1, ln_g1, ln_b1):
    row = edge_index[0]
    col = edge_index[1]
    e0 = NS * 4 * NIT0 * CB
    e1cap = NS * 4 * NIT1 * CB
    pad = jnp.full((e0 + e1cap - E,), N, dtype=jnp.int32)

    def _split(v):
        p0 = v[:e0].reshape(NS, 4 * NIT0, CB)
        p1 = jnp.concatenate([v[e0:], pad]).reshape(NS, 4 * NIT1, CB)
        p1 = jnp.pad(p1, ((0, 0), (0, NCHMAX - 4 * NIT1), (0, 0)))
        return jnp.concatenate([p0, p1], axis=0)

    row_p = _split(row)
    col_p = _split(col)
    zeros_hbm = jnp.zeros((NP, DW), jnp.float32)
    tab_pad = jnp.full((NP - N,), -30.0, jnp.float32)

    layers = [
        (ws_w0, ws_b0, wn_w0, wn_b0, at_w0, at_b0, ln_g0, ln_b0),
        (ws_w1, ws_b1, wn_w1, wn_b1, at_w1, at_b1, ln_g1, ln_b1),
    ]
    for (ws_w, ws_b, wn_w, wn_b, at_w, at_b, ln_g, ln_b) in layers:
        xs, xne, ab = _dense(x, ws_w, ws_b, wn_w, wn_b, at_w, at_b)
        xne_p = jnp.concatenate(
            [xne, jnp.zeros((NP - N, DW), jnp.float32)], axis=0)
        a_p = jnp.concatenate([ab[:, 0], tab_pad])
        parts = _sc_agg(xne_p, row_p, col_p, a_p, zeros_hbm)
        x = _epilogue(x, xs, parts[0], parts[1], ln_g, ln_b)
    return x


# speedup vs baseline: 1.0791x; 1.0791x over previous
import functools

import jax
import jax.numpy as jnp
from jax import lax
from jax.experimental import pallas as pl
from jax.experimental.pallas import tpu as pltpu
from jax.experimental.pallas import tpu_sc as plsc

N = 10000
D = 128
E = 320000
NC, NS, L = 2, 16, 16
NW = NC * NS
DW = 144
NP = 10016
CB = 64
NIT0 = 51
NIT1 = 28
NCHMAX = 4 * NIT0
RPT = NP // NS
BN = 2000


def _dense_body(x_ref, wsw_ref, wsb_ref, wnw_ref, wnb_ref, atw_ref, atb_ref,
                xs_ref, xne_ref, ab_ref):
    xb = x_ref[...]
    xs_ref[...] = jnp.dot(xb, wsw_ref[...],
                          preferred_element_type=jnp.float32) + wsb_ref[...]
    xn = jnp.dot(xb, wnw_ref[...],
                 preferred_element_type=jnp.float32) + wnb_ref[...]
    atw = atw_ref[...]
    a = jnp.dot(xb, atw[:D, :], preferred_element_type=jnp.float32)
    b = jnp.dot(xn, atw[D:, :], preferred_element_type=jnp.float32) \
        + atb_ref[...]
    m = x_ref.shape[0]
    xne_ref[...] = jnp.concatenate(
        [xn, jnp.ones((m, 1), jnp.float32), b,
         jnp.zeros((m, DW - D - 2), jnp.float32)], axis=-1)
    ab_ref[...] = jnp.concatenate([a, b], axis=-1)


def _dense(x, ws_w, ws_b, wn_w, wn_b, at_w, at_b):
    grid = (N // BN,)
    return pl.pallas_call(
        _dense_body,
        grid=grid,
        in_specs=[
            pl.BlockSpec((BN, D), lambda i: (i, 0)),
            pl.BlockSpec((D, D), lambda i: (0, 0)),
            pl.BlockSpec((D,), lambda i: (0,)),
            pl.BlockSpec((D, D), lambda i: (0, 0)),
            pl.BlockSpec((D,), lambda i: (0,)),
            pl.BlockSpec((2 * D, 1), lambda i: (0, 0)),
            pl.BlockSpec((1,), lambda i: (0,)),
        ],
        out_specs=[
            pl.BlockSpec((BN, D), lambda i: (i, 0)),
            pl.BlockSpec((BN, DW), lambda i: (i, 0)),
            pl.BlockSpec((BN, 2), lambda i: (i, 0)),
        ],
        out_shape=[
            jax.ShapeDtypeStruct((N, D), jnp.float32),
            jax.ShapeDtypeStruct((N, DW), jnp.float32),
            jax.ShapeDtypeStruct((N, 2), jnp.float32),
        ],
    )(x, ws_w, ws_b, wn_w, wn_b, at_w, at_b)


def _sc_body(xne_hbm, row_hbm, col_hbm, a_hbm, z_hbm, out_hbm,
             rowc0, colc0, rowc1, colc1, rowc2, colc2, rowc3, colc3,
             av, rv0, rv1, aggsh,
             sem_i0, sem_i1, sem_i2, sem_i3, sem_g0, sem_g1,
             sem_s0, sem_s1):
    cid = lax.axis_index("c")
    sid = lax.axis_index("s")
    wid = cid * NS + sid
    rowc = [rowc0, rowc1, rowc2, rowc3]
    colc = [colc0, colc1, colc2, colc3]
    rv = [rv0, rv1]
    sem_i = [sem_i0, sem_i1, sem_i2, sem_i3]
    sem_g = [sem_g0, sem_g1]
    sem_s = [sem_s0, sem_s1]
    rhb = row_hbm.at[wid]
    chb = col_hbm.at[wid]
    nit = jnp.where(cid == 0, NIT0, NIT1)

    pltpu.sync_copy(a_hbm, av)
    pltpu.sync_copy(z_hbm.at[pl.ds(sid * RPT, RPT)],
                    aggsh.at[pl.ds(sid * RPT, RPT)])
    plsc.subcore_barrier()

    pltpu.sync_copy(rhb.at[0], rowc[0])
    pltpu.sync_copy(chb.at[0], colc[0])
    pltpu.async_copy(rhb.at[1], rowc[1], sem_i[1])
    pltpu.async_copy(chb.at[1], colc[1], sem_i[1])
    pltpu.async_copy(xne_hbm.at[rowc[0]], rv[0], sem_g[0])

    def compute(q, p):
        def group(g, cy):
            c16 = colc[q][pl.ds(g * L, L)]
            jrow = g * L + lax.iota(jnp.int32, L)
            jcol = jnp.full((L,), D + 1, jnp.int32)
            b16 = plsc.load_gather(rv[p], [jrow, jcol])
            z = plsc.load_gather(av, [c16]) + b16
            s = 1.0 / (1.0 + jnp.exp(-z))
            for jl in range(L):
                j = g * L + jl
                sj = s[jl]
                for u in range(DW // L):
                    rv[p][j, pl.ds(u * L, L)] = rv[p][j, pl.ds(u * L, L)] * sj
            return cy
        lax.fori_loop(0, CB // L, group, 0)

    def iter_body(m, cy):
        for r in range(4):
            c = 4 * m + r
            q, p = r, r % 2
            qn1, qn2, qm1 = (r + 1) % 4, (r + 2) % 4, (r + 3) % 4

            def fire_idx(c=c, qn2=qn2):
                pltpu.async_copy(rhb.at[c + 2], rowc[qn2], sem_i[qn2])
                pltpu.async_copy(chb.at[c + 2], colc[qn2], sem_i[qn2])

            if r >= 2:
                pl.when(m < nit - 1)(fire_idx)
            else:
                fire_idx()

            pltpu.make_async_copy(xne_hbm.at[rowc[q]], rv[p], sem_g[p]).wait()
            compute(q, p)

            def wait_sc(p=p, qm1=qm1):
                pltpu.make_async_copy(rv[1 - p], aggsh.at[colc[qm1]],
                                      sem_s[1 - p]).wait()

            if r == 0:
                pl.when(m > 0)(wait_sc)
            else:
                wait_sc()

            def fire_g(c=c, qn1=qn1, p=p):
                pltpu.make_async_copy(rhb.at[c + 1], rowc[qn1],
                                      sem_i[qn1]).wait()
                pltpu.make_async_copy(chb.at[c + 1], colc[qn1],
                                      sem_i[qn1]).wait()
                pltpu.async_copy(xne_hbm.at[rowc[qn1]], rv[1 - p],
                                 sem_g[1 - p])

            if r == 3:
                pl.when(m < nit - 1)(fire_g)
            else:
                fire_g()

            pltpu.async_copy(rv[p], aggsh.at[colc[q]], sem_s[p], add=True)
        return cy

    lax.fori_loop(0, nit, iter_body, 0)
    pltpu.make_async_copy(rv[1], aggsh.at[colc[3]], sem_s[1]).wait()
    plsc.subcore_barrier()
    pltpu.sync_copy(aggsh.at[pl.ds(sid * RPT, RPT)],
                    out_hbm.at[cid].at[pl.ds(sid * RPT, RPT)])


_sc_agg = functools.partial(
    pl.kernel,
    _sc_body,
    out_type=jax.ShapeDtypeStruct((NC, NP, DW), jnp.float32),
    mesh=plsc.VectorSubcoreMesh(core_axis_name="c", subcore_axis_name="s"),
    scratch_types=[
        pltpu.VMEM((CB,), jnp.int32),
        pltpu.VMEM((CB,), jnp.int32),
        pltpu.VMEM((CB,), jnp.int32),
        pltpu.VMEM((CB,), jnp.int32),
        pltpu.VMEM((CB,), jnp.int32),
        pltpu.VMEM((CB,), jnp.int32),
        pltpu.VMEM((CB,), jnp.int32),
        pltpu.VMEM((CB,), jnp.int32),
        pltpu.VMEM((NP,), jnp.float32),
        pltpu.VMEM((CB, DW), jnp.float32),
        pltpu.VMEM((CB, DW), jnp.float32),
        pltpu.VMEM_SHARED((NP, DW), jnp.float32),
        pltpu.SemaphoreType.DMA,
        pltpu.SemaphoreType.DMA,
        pltpu.SemaphoreType.DMA,
        pltpu.SemaphoreType.DMA,
        pltpu.SemaphoreType.DMA,
        pltpu.SemaphoreType.DMA,
        pltpu.SemaphoreType.DMA,
        pltpu.SemaphoreType.DMA,
    ],
    compiler_params=pltpu.CompilerParams(needs_layout_passes=False,
                                         use_tc_tiling_on_sc=False),
)()


def _epi_body(x_ref, xs_ref, p0_ref, p1_ref, g_ref, bb_ref, o_ref):
    agg = p0_ref[:, :D] + p1_ref[:, :D]
    asum = p0_ref[:, D:D + 1] + p1_ref[:, D:D + 1]
    u = xs_ref[...] + agg / (asum + 1e-6)
    h = 0.5 * u * (1.0 + lax.erf(u * (1.0 / jnp.sqrt(2.0)))) + x_ref[...]
    m = jnp.mean(h, axis=-1, keepdims=True)
    v = jnp.mean((h - m) ** 2, axis=-1, keepdims=True)
    o_ref[...] = (h - m) / jnp.sqrt(v + 1e-5) * g_ref[...] + bb_ref[...]


def _epilogue(x, xs, p0, p1, ln_g, ln_b):
    grid = (N // BN,)
    return pl.pallas_call(
        _epi_body,
        grid=grid,
        in_specs=[
            pl.BlockSpec((BN, D), lambda i: (i, 0)),
            pl.BlockSpec((BN, D), lambda i: (i, 0)),
            pl.BlockSpec((BN, DW), lambda i: (i, 0)),
            pl.BlockSpec((BN, DW), lambda i: (i, 0)),
            pl.BlockSpec((D,), lambda i: (0,)),
            pl.BlockSpec((D,), lambda i: (0,)),
        ],
        out_specs=pl.BlockSpec((BN, D), lambda i: (i, 0)),
        out_shape=jax.ShapeDtypeStruct((N, D), jnp.float32),
    )(x, xs, p0, p1, ln_g, ln_b)


def kernel(x, edge_index, ws_w0, ws_b0, wn_w0, wn_b0, at_w0, at_b0, ln_g0,
           ln_b0, ws_w1, ws_b1, wn_w1, wn_b1, at_w1, at_b1, ln_g1, ln_b1):
    row = edge_index[0]
    col = edge_index[1]
    e0 = NS * 4 * NIT0 * CB
    e1cap = NS * 4 * NIT1 * CB
    pad = jnp.full((e0 + e1cap - E,), N, dtype=jnp.int32)

    def _split(v):
        p0 = v[:e0].reshape(NS, 4 * NIT0, CB)
        p1 = jnp.concatenate([v[e0:], pad]).reshape(NS, 4 * NIT1, CB)
        p1 = jnp.pad(p1, ((0, 0), (0, NCHMAX - 4 * NIT1), (0, 0)))
        return jnp.concatenate([p0, p1], axis=0)

    row_p = _split(row)
    col_p = _split(col)
    zeros_hbm = jnp.zeros((NP, DW), jnp.float32)
    tab_pad = jnp.full((NP - N,), -30.0, jnp.float32)

    layers = [
        (ws_w0, ws_b0, wn_w0, wn_b0, at_w0, at_b0, ln_g0, ln_b0),
        (ws_w1, ws_b1, wn_w1, wn_b1, at_w1, at_b1, ln_g1, ln_b1),
    ]
    for (ws_w, ws_b, wn_w, wn_b, at_w, at_b, ln_g, ln_b) in layers:
        xs, xne, ab = _dense(x, ws_w, ws_b, wn_w, wn_b, at_w, at_b)
        xne_p = jnp.concatenate(
            [xne, jnp.zeros((NP - N, DW), jnp.float32)], axis=0)
        a_p = jnp.concatenate([ab[:, 0], tab_pad])
        parts = _sc_agg(xne_p, row_p, col_p, a_p, zeros_hbm)
        x = _epilogue(x, xs, parts[0], parts[1], ln_g, ln_b)
    return x


# speedup vs baseline: 1.1229x; 1.0406x over previous
import functools

import jax
import jax.numpy as jnp
from jax import lax
from jax.experimental import pallas as pl
from jax.experimental.pallas import tpu as pltpu
from jax.experimental.pallas import tpu_sc as plsc

N = 10000
D = 128
E = 320000
NC, NS, L = 2, 16, 16
NW = NC * NS
DW = 144
NP = 10016
CB = 64
NIT0 = 49
NIT1 = 30
NCHMAX = 4 * NIT0
RPT = NP // NS
BN = 2000


def _dense_body(x_ref, wsw_ref, wsb_ref, wnw_ref, wnb_ref, atw_ref, atb_ref,
                xs_ref, xne_ref, ab_ref):
    xb = x_ref[...]
    xs_ref[...] = jnp.dot(xb, wsw_ref[...],
                          preferred_element_type=jnp.float32) + wsb_ref[...]
    xn = jnp.dot(xb, wnw_ref[...],
                 preferred_element_type=jnp.float32) + wnb_ref[...]
    atw = atw_ref[...]
    a = jnp.dot(xb, atw[:D, :], preferred_element_type=jnp.float32)
    b = jnp.dot(xn, atw[D:, :], preferred_element_type=jnp.float32) \
        + atb_ref[...]
    m = x_ref.shape[0]
    xne_ref[...] = jnp.concatenate(
        [xn, jnp.ones((m, 1), jnp.float32), b,
         jnp.zeros((m, DW - D - 2), jnp.float32)], axis=-1)
    ab_ref[...] = jnp.concatenate([a, b], axis=-1)


def _dense(x, ws_w, ws_b, wn_w, wn_b, at_w, at_b):
    grid = (N // BN,)
    return pl.pallas_call(
        _dense_body,
        grid=grid,
        in_specs=[
            pl.BlockSpec((BN, D), lambda i: (i, 0)),
            pl.BlockSpec((D, D), lambda i: (0, 0)),
            pl.BlockSpec((D,), lambda i: (0,)),
            pl.BlockSpec((D, D), lambda i: (0, 0)),
            pl.BlockSpec((D,), lambda i: (0,)),
            pl.BlockSpec((2 * D, 1), lambda i: (0, 0)),
            pl.BlockSpec((1,), lambda i: (0,)),
        ],
        out_specs=[
            pl.BlockSpec((BN, D), lambda i: (i, 0)),
            pl.BlockSpec((BN, DW), lambda i: (i, 0)),
            pl.BlockSpec((BN, 2), lambda i: (i, 0)),
        ],
        out_shape=[
            jax.ShapeDtypeStruct((N, D), jnp.float32),
            jax.ShapeDtypeStruct((N, DW), jnp.float32),
            jax.ShapeDtypeStruct((N, 2), jnp.float32),
        ],
    )(x, ws_w, ws_b, wn_w, wn_b, at_w, at_b)


def _sc_body(xne_hbm, row_hbm, col_hbm, a_hbm, out_hbm,
             rowc0, colc0, rowc1, colc1, rowc2, colc2, rowc3, colc3,
             av, rv0, rv1, aggsh,
             sem_i0, sem_i1, sem_i2, sem_i3, sem_g0, sem_g1,
             sem_s0, sem_s1):
    cid = lax.axis_index("c")
    sid = lax.axis_index("s")
    wid = cid * NS + sid
    rowc = [rowc0, rowc1, rowc2, rowc3]
    colc = [colc0, colc1, colc2, colc3]
    rv = [rv0, rv1]
    sem_i = [sem_i0, sem_i1, sem_i2, sem_i3]
    sem_g = [sem_g0, sem_g1]
    sem_s = [sem_s0, sem_s1]
    rhb = row_hbm.at[wid]
    chb = col_hbm.at[wid]
    nit = jnp.where(cid == 0, NIT0, NIT1)

    pltpu.sync_copy(a_hbm, av)

    def zrow(j, cy):
        for u in range(DW // L):
            rv0[j, pl.ds(u * L, L)] = jnp.zeros((L,), jnp.float32)
        return cy
    lax.fori_loop(0, CB, zrow, 0)
    base = sid * RPT
    for k in range(RPT // CB):
        pltpu.sync_copy(rv0, aggsh.at[pl.ds(base + k * CB, CB)])
    rem = RPT % CB
    if rem:
        pltpu.sync_copy(rv0.at[pl.ds(0, rem)],
                        aggsh.at[pl.ds(base + (RPT // CB) * CB, rem)])
    plsc.subcore_barrier()

    pltpu.sync_copy(rhb.at[0], rowc[0])
    pltpu.sync_copy(chb.at[0], colc[0])
    pltpu.async_copy(rhb.at[1], rowc[1], sem_i[1])
    pltpu.async_copy(chb.at[1], colc[1], sem_i[1])
    pltpu.async_copy(xne_hbm.at[rowc[0]], rv[0], sem_g[0])

    def compute(q, p):
        def group(g, cy):
            c16 = colc[q][pl.ds(g * L, L)]
            jrow = g * L + lax.iota(jnp.int32, L)
            jcol = jnp.full((L,), D + 1, jnp.int32)
            b16 = plsc.load_gather(rv[p], [jrow, jcol])
            z = plsc.load_gather(av, [c16]) + b16
            s = 1.0 / (1.0 + jnp.exp(-z))
            for jl in range(L):
                j = g * L + jl
                sj = s[jl]
                for u in range(DW // L):
                    rv[p][j, pl.ds(u * L, L)] = rv[p][j, pl.ds(u * L, L)] * sj
            return cy
        lax.fori_loop(0, CB // L, group, 0)

    def iter_body(m, cy):
        for r in range(4):
            c = 4 * m + r
            q, p = r, r % 2
            qn1, qn2, qm1 = (r + 1) % 4, (r + 2) % 4, (r + 3) % 4

            def fire_idx(c=c, qn2=qn2):
                pltpu.async_copy(rhb.at[c + 2], rowc[qn2], sem_i[qn2])
                pltpu.async_copy(chb.at[c + 2], colc[qn2], sem_i[qn2])

            if r >= 2:
                pl.when(m < nit - 1)(fire_idx)
            else:
                fire_idx()

            pltpu.make_async_copy(xne_hbm.at[rowc[q]], rv[p], sem_g[p]).wait()
            compute(q, p)

            def wait_sc(p=p, qm1=qm1):
                pltpu.make_async_copy(rv[1 - p], aggsh.at[colc[qm1]],
                                      sem_s[1 - p]).wait()

            if r == 0:
                pl.when(m > 0)(wait_sc)
            else:
                wait_sc()

            def fire_g(c=c, qn1=qn1, p=p):
                pltpu.make_async_copy(rhb.at[c + 1], rowc[qn1],
                                      sem_i[qn1]).wait()
                pltpu.make_async_copy(chb.at[c + 1], colc[qn1],
                                      sem_i[qn1]).wait()
                pltpu.async_copy(xne_hbm.at[rowc[qn1]], rv[1 - p],
                                 sem_g[1 - p])

            if r == 3:
                pl.when(m < nit - 1)(fire_g)
            else:
                fire_g()

            pltpu.async_copy(rv[p], aggsh.at[colc[q]], sem_s[p], add=True)
        return cy

    lax.fori_loop(0, nit, iter_body, 0)
    pltpu.make_async_copy(rv[1], aggsh.at[colc[3]], sem_s[1]).wait()
    plsc.subcore_barrier()
    pltpu.sync_copy(aggsh.at[pl.ds(sid * RPT, RPT)],
                    out_hbm.at[cid].at[pl.ds(sid * RPT, RPT)])


_sc_agg = functools.partial(
    pl.kernel,
    _sc_body,
    out_type=jax.ShapeDtypeStruct((NC, NP, DW), jnp.float32),
    mesh=plsc.VectorSubcoreMesh(core_axis_name="c", subcore_axis_name="s"),
    scratch_types=[
        pltpu.VMEM((CB,), jnp.int32),
        pltpu.VMEM((CB,), jnp.int32),
        pltpu.VMEM((CB,), jnp.int32),
        pltpu.VMEM((CB,), jnp.int32),
        pltpu.VMEM((CB,), jnp.int32),
        pltpu.VMEM((CB,), jnp.int32),
        pltpu.VMEM((CB,), jnp.int32),
        pltpu.VMEM((CB,), jnp.int32),
        pltpu.VMEM((NP,), jnp.float32),
        pltpu.VMEM((CB, DW), jnp.float32),
        pltpu.VMEM((CB, DW), jnp.float32),
        pltpu.VMEM_SHARED((NP, DW), jnp.float32),
        pltpu.SemaphoreType.DMA,
        pltpu.SemaphoreType.DMA,
        pltpu.SemaphoreType.DMA,
        pltpu.SemaphoreType.DMA,
        pltpu.SemaphoreType.DMA,
        pltpu.SemaphoreType.DMA,
        pltpu.SemaphoreType.DMA,
        pltpu.SemaphoreType.DMA,
    ],
    compiler_params=pltpu.CompilerParams(needs_layout_passes=False,
                                         use_tc_tiling_on_sc=False),
)()


def _epi_body(x_ref, xs_ref, p0_ref, p1_ref, g_ref, bb_ref, o_ref):
    agg = p0_ref[:, :D] + p1_ref[:, :D]
    asum = p0_ref[:, D:D + 1] + p1_ref[:, D:D + 1]
    u = xs_ref[...] + agg / (asum + 1e-6)
    h = 0.5 * u * (1.0 + lax.erf(u * (1.0 / jnp.sqrt(2.0)))) + x_ref[...]
    m = jnp.mean(h, axis=-1, keepdims=True)
    v = jnp.mean((h - m) ** 2, axis=-1, keepdims=True)
    o_ref[...] = (h - m) / jnp.sqrt(v + 1e-5) * g_ref[...] + bb_ref[...]


def _epilogue(x, xs, p0, p1, ln_g, ln_b):
    grid = (N // BN,)
    return pl.pallas_call(
        _epi_body,
        grid=grid,
        in_specs=[
            pl.BlockSpec((BN, D), lambda i: (i, 0)),
            pl.BlockSpec((BN, D), lambda i: (i, 0)),
            pl.BlockSpec((BN, DW), lambda i: (i, 0)),
            pl.BlockSpec((BN, DW), lambda i: (i, 0)),
            pl.BlockSpec((D,), lambda i: (0,)),
            pl.BlockSpec((D,), lambda i: (0,)),
        ],
        out_specs=pl.BlockSpec((BN, D), lambda i: (i, 0)),
        out_shape=jax.ShapeDtypeStruct((N, D), jnp.float32),
    )(x, xs, p0, p1, ln_g, ln_b)


def kernel(x, edge_index, ws_w0, ws_b0, wn_w0, wn_b0, at_w0, at_b0, ln_g0,
           ln_b0, ws_w1, ws_b1, wn_w1, wn_b1, at_w1, at_b1, ln_g1, ln_b1):
    row = edge_index[0]
    col = edge_index[1]
    e0 = NS * 4 * NIT0 * CB
    e1cap = NS * 4 * NIT1 * CB
    pad = jnp.full((e0 + e1cap - E,), N, dtype=jnp.int32)

    def _split(v):
        p0 = v[:e0].reshape(NS, 4 * NIT0, CB)
        p1 = jnp.concatenate([v[e0:], pad]).reshape(NS, 4 * NIT1, CB)
        p1 = jnp.pad(p1, ((0, 0), (0, NCHMAX - 4 * NIT1), (0, 0)))
        return jnp.concatenate([p0, p1], axis=0)

    row_p = _split(row)
    col_p = _split(col)
    tab_pad = jnp.full((NP - N,), -30.0, jnp.float32)

    layers = [
        (ws_w0, ws_b0, wn_w0, wn_b0, at_w0, at_b0, ln_g0, ln_b0),
        (ws_w1, ws_b1, wn_w1, wn_b1, at_w1, at_b1, ln_g1, ln_b1),
    ]
    for (ws_w, ws_b, wn_w, wn_b, at_w, at_b, ln_g, ln_b) in layers:
        xs, xne, ab = _dense(x, ws_w, ws_b, wn_w, wn_b, at_w, at_b)
        xne_p = jnp.concatenate(
            [xne, jnp.zeros((NP - N, DW), jnp.float32)], axis=0)
        a_p = jnp.concatenate([ab[:, 0], tab_pad])
        parts = _sc_agg(xne_p, row_p, col_p, a_p)
        x = _epilogue(x, xs, parts[0], parts[1], ln_g, ln_b)
    return x


# speedup vs baseline: 1.1234x; 1.0005x over previous
import functools

import jax
import jax.numpy as jnp
from jax import lax
from jax.experimental import pallas as pl
from jax.experimental.pallas import tpu as pltpu
from jax.experimental.pallas import tpu_sc as plsc

N = 10000
D = 128
E = 320000
NC, NS, L = 2, 16, 16
NW = NC * NS
DW = 144
NP = 10016
CB = 64
NIT0 = 49
NIT1 = 30
NCHMAX = 4 * NIT0
RPT = NP // NS
BN = 2000


def _dense_body(x_ref, wsw_ref, wsb_ref, wnw_ref, wnb_ref, atw_ref, atb_ref,
                xs_ref, xne_ref, ab_ref):
    xb = x_ref[...]
    xs_ref[...] = jnp.dot(xb, wsw_ref[...],
                          preferred_element_type=jnp.float32) + wsb_ref[...]
    xn = jnp.dot(xb, wnw_ref[...],
                 preferred_element_type=jnp.float32) + wnb_ref[...]
    atw = atw_ref[...]
    a = jnp.dot(xb, atw[:D, :], preferred_element_type=jnp.float32)
    b = jnp.dot(xn, atw[D:, :], preferred_element_type=jnp.float32) \
        + atb_ref[...]
    m = x_ref.shape[0]
    xne_ref[...] = jnp.concatenate(
        [xn, jnp.ones((m, 1), jnp.float32), b,
         jnp.zeros((m, DW - D - 2), jnp.float32)], axis=-1)
    ab_ref[...] = jnp.concatenate([a, b], axis=-1)


def _dense(x, ws_w, ws_b, wn_w, wn_b, at_w, at_b):
    grid = (N // BN,)
    return pl.pallas_call(
        _dense_body,
        grid=grid,
        in_specs=[
            pl.BlockSpec((BN, D), lambda i: (i, 0)),
            pl.BlockSpec((D, D), lambda i: (0, 0)),
            pl.BlockSpec((D,), lambda i: (0,)),
            pl.BlockSpec((D, D), lambda i: (0, 0)),
            pl.BlockSpec((D,), lambda i: (0,)),
            pl.BlockSpec((2 * D, 1), lambda i: (0, 0)),
            pl.BlockSpec((1,), lambda i: (0,)),
        ],
        out_specs=[
            pl.BlockSpec((BN, D), lambda i: (i, 0)),
            pl.BlockSpec((BN, DW), lambda i: (i, 0)),
            pl.BlockSpec((BN, 2), lambda i: (i, 0)),
        ],
        out_shape=[
            jax.ShapeDtypeStruct((N, D), jnp.float32),
            jax.ShapeDtypeStruct((N, DW), jnp.float32),
            jax.ShapeDtypeStruct((N, 2), jnp.float32),
        ],
    )(x, ws_w, ws_b, wn_w, wn_b, at_w, at_b)


def _sc_body(xne_hbm, row_hbm, col_hbm, a_hbm, out_hbm,
             rowc0, colc0, rowc1, colc1, rowc2, colc2, rowc3, colc3,
             av, rv0, rv1, aggsh,
             sem_i0, sem_i1, sem_i2, sem_i3, sem_g0, sem_g1,
             sem_s0, sem_s1):
    cid = lax.axis_index("c")
    sid = lax.axis_index("s")
    wid = cid * NS + sid
    rowc = [rowc0, rowc1, rowc2, rowc3]
    colc = [colc0, colc1, colc2, colc3]
    rv = [rv0, rv1]
    sem_i = [sem_i0, sem_i1, sem_i2, sem_i3]
    sem_g = [sem_g0, sem_g1]
    sem_s = [sem_s0, sem_s1]
    rhb = row_hbm.at[wid]
    chb = col_hbm.at[wid]
    nit = jnp.where(cid == 0, NIT0, NIT1)

    pltpu.sync_copy(a_hbm, av)

    def zrow(j, cy):
        for u in range(DW // L):
            rv0[j, pl.ds(u * L, L)] = jnp.zeros((L,), jnp.float32)
        return cy
    lax.fori_loop(0, CB, zrow, 0)
    base = sid * RPT
    for k in range(RPT // CB):
        pltpu.sync_copy(rv0, aggsh.at[pl.ds(base + k * CB, CB)])
    rem = RPT % CB
    if rem:
        pltpu.sync_copy(rv0.at[pl.ds(0, rem)],
                        aggsh.at[pl.ds(base + (RPT // CB) * CB, rem)])
    plsc.subcore_barrier()

    HB = CB // 2

    def fire_gather(q, p):
        pltpu.async_copy(xne_hbm.at[rowc[q].at[pl.ds(0, HB)]],
                         rv[p].at[pl.ds(0, HB)], sem_g[p])
        pltpu.async_copy(xne_hbm.at[rowc[q].at[pl.ds(HB, HB)]],
                         rv[p].at[pl.ds(HB, HB)], sem_g[p])

    def wait_gather(q, p):
        pltpu.make_async_copy(xne_hbm.at[rowc[q].at[pl.ds(0, HB)]],
                              rv[p].at[pl.ds(0, HB)], sem_g[p]).wait()
        pltpu.make_async_copy(xne_hbm.at[rowc[q].at[pl.ds(HB, HB)]],
                              rv[p].at[pl.ds(HB, HB)], sem_g[p]).wait()

    pltpu.sync_copy(rhb.at[0], rowc[0])
    pltpu.sync_copy(chb.at[0], colc[0])
    pltpu.async_copy(rhb.at[1], rowc[1], sem_i[1])
    pltpu.async_copy(chb.at[1], colc[1], sem_i[1])
    fire_gather(0, 0)

    def compute(q, p):
        def group(g, cy):
            c16 = colc[q][pl.ds(g * L, L)]
            jrow = g * L + lax.iota(jnp.int32, L)
            jcol = jnp.full((L,), D + 1, jnp.int32)
            b16 = plsc.load_gather(rv[p], [jrow, jcol])
            z = plsc.load_gather(av, [c16]) + b16
            s = 1.0 / (1.0 + jnp.exp(-z))
            for jl in range(L):
                j = g * L + jl
                sj = s[jl]
                for u in range(DW // L):
                    rv[p][j, pl.ds(u * L, L)] = rv[p][j, pl.ds(u * L, L)] * sj
            return cy
        lax.fori_loop(0, CB // L, group, 0)

    def iter_body(m, cy):
        for r in range(4):
            c = 4 * m + r
            q, p = r, r % 2
            qn1, qn2, qm1 = (r + 1) % 4, (r + 2) % 4, (r + 3) % 4

            def fire_idx(c=c, qn2=qn2):
                pltpu.async_copy(rhb.at[c + 2], rowc[qn2], sem_i[qn2])
                pltpu.async_copy(chb.at[c + 2], colc[qn2], sem_i[qn2])

            if r >= 2:
                pl.when(m < nit - 1)(fire_idx)
            else:
                fire_idx()

            wait_gather(q, p)
            compute(q, p)

            def wait_sc(p=p, qm1=qm1):
                pltpu.make_async_copy(rv[1 - p], aggsh.at[colc[qm1]],
                                      sem_s[1 - p]).wait()

            if r == 0:
                pl.when(m > 0)(wait_sc)
            else:
                wait_sc()

            def fire_g(c=c, qn1=qn1, p=p):
                pltpu.make_async_copy(rhb.at[c + 1], rowc[qn1],
                                      sem_i[qn1]).wait()
                pltpu.make_async_copy(chb.at[c + 1], colc[qn1],
                                      sem_i[qn1]).wait()
                fire_gather(qn1, 1 - p)

            if r == 3:
                pl.when(m < nit - 1)(fire_g)
            else:
                fire_g()

            pltpu.async_copy(rv[p], aggsh.at[colc[q]], sem_s[p], add=True)
        return cy

    lax.fori_loop(0, nit, iter_body, 0)
    pltpu.make_async_copy(rv[1], aggsh.at[colc[3]], sem_s[1]).wait()
    plsc.subcore_barrier()
    pltpu.sync_copy(aggsh.at[pl.ds(sid * RPT, RPT)],
                    out_hbm.at[cid].at[pl.ds(sid * RPT, RPT)])


_sc_agg = functools.partial(
    pl.kernel,
    _sc_body,
    out_type=jax.ShapeDtypeStruct((NC, NP, DW), jnp.float32),
    mesh=plsc.VectorSubcoreMesh(core_axis_name="c", subcore_axis_name="s"),
    scratch_types=[
        pltpu.VMEM((CB,), jnp.int32),
        pltpu.VMEM((CB,), jnp.int32),
        pltpu.VMEM((CB,), jnp.int32),
        pltpu.VMEM((CB,), jnp.int32),
        pltpu.VMEM((CB,), jnp.int32),
        pltpu.VMEM((CB,), jnp.int32),
        pltpu.VMEM((CB,), jnp.int32),
        pltpu.VMEM((CB,), jnp.int32),
        pltpu.VMEM((NP,), jnp.float32),
        pltpu.VMEM((CB, DW), jnp.float32),
        pltpu.VMEM((CB, DW), jnp.float32),
        pltpu.VMEM_SHARED((NP, DW), jnp.float32),
        pltpu.SemaphoreType.DMA,
        pltpu.SemaphoreType.DMA,
        pltpu.SemaphoreType.DMA,
        pltpu.SemaphoreType.DMA,
        pltpu.SemaphoreType.DMA,
        pltpu.SemaphoreType.DMA,
        pltpu.SemaphoreType.DMA,
        pltpu.SemaphoreType.DMA,
    ],
    compiler_params=pltpu.CompilerParams(needs_layout_passes=False,
                                         use_tc_tiling_on_sc=False),
)()


def _epi_body(x_ref, xs_ref, p0_ref, p1_ref, g_ref, bb_ref, o_ref):
    agg = p0_ref[:, :D] + p1_ref[:, :D]
    asum = p0_ref[:, D:D + 1] + p1_ref[:, D:D + 1]
    u = xs_ref[...] + agg / (asum + 1e-6)
    h = 0.5 * u * (1.0 + lax.erf(u * (1.0 / jnp.sqrt(2.0)))) + x_ref[...]
    m = jnp.mean(h, axis=-1, keepdims=True)
    v = jnp.mean((h - m) ** 2, axis=-1, keepdims=True)
    o_ref[...] = (h - m) / jnp.sqrt(v + 1e-5) * g_ref[...] + bb_ref[...]


def _epilogue(x, xs, p0, p1, ln_g, ln_b):
    grid = (N // BN,)
    return pl.pallas_call(
        _epi_body,
        grid=grid,
        in_specs=[
            pl.BlockSpec((BN, D), lambda i: (i, 0)),
            pl.BlockSpec((BN, D), lambda i: (i, 0)),
            pl.BlockSpec((BN, DW), lambda i: (i, 0)),
            pl.BlockSpec((BN, DW), lambda i: (i, 0)),
            pl.BlockSpec((D,), lambda i: (0,)),
            pl.BlockSpec((D,), lambda i: (0,)),
        ],
        out_specs=pl.BlockSpec((BN, D), lambda i: (i, 0)),
        out_shape=jax.ShapeDtypeStruct((N, D), jnp.float32),
    )(x, xs, p0, p1, ln_g, ln_b)


def kernel(x, edge_index, ws_w0, ws_b0, wn_w0, wn_b0, at_w0, at_b0, ln_g0,
           ln_b0, ws_w1, ws_b1, wn_w1, wn_b1, at_w1, at_b1, ln_g1, ln_b1):
    row = edge_index[0]
    col = edge_index[1]
    e0 = NS * 4 * NIT0 * CB
    e1cap = NS * 4 * NIT1 * CB
    pad = jnp.full((e0 + e1cap - E,), N, dtype=jnp.int32)

    def _split(v):
        p0 = v[:e0].reshape(NS, 4 * NIT0, CB)
        p1 = jnp.concatenate([v[e0:], pad]).reshape(NS, 4 * NIT1, CB)
        p1 = jnp.pad(p1, ((0, 0), (0, NCHMAX - 4 * NIT1), (0, 0)))
        return jnp.concatenate([p0, p1], axis=0)

    row_p = _split(row)
    col_p = _split(col)
    tab_pad = jnp.full((NP - N,), -30.0, jnp.float32)

    layers = [
        (ws_w0, ws_b0, wn_w0, wn_b0, at_w0, at_b0, ln_g0, ln_b0),
        (ws_w1, ws_b1, wn_w1, wn_b1, at_w1, at_b1, ln_g1, ln_b1),
    ]
    for (ws_w, ws_b, wn_w, wn_b, at_w, at_b, ln_g, ln_b) in layers:
        xs, xne, ab = _dense(x, ws_w, ws_b, wn_w, wn_b, at_w, at_b)
        xne_p = jnp.concatenate(
            [xne, jnp.zeros((NP - N, DW), jnp.float32)], axis=0)
        a_p = jnp.concatenate([ab[:, 0], tab_pad])
        parts = _sc_agg(xne_p, row_p, col_p, a_p)
        x = _epilogue(x, xs, parts[0], parts[1], ln_g, ln_b)
    return x
